# baseline (speedup 1.0000x reference)
import functools

import jax
import jax.numpy as jnp
from jax import lax
from jax.experimental import pallas as pl
from jax.experimental.pallas import tpu as pltpu
from jax.experimental.pallas import tpu_sc as plsc

_NC, _NS, _L = 2, 16, 16
_NW = _NC * _NS
_C = 1048576
_NROWS = 12

_CSC = _C // 2
_CTC = _C - _CSC
_CT = 32768
_NT = _CTC // _CT
_TCR = 2048

_CW = _CSC // _NW
_CH = _CW // 2
_NQ = 2 * _NROWS
_NB = 4
assert _CH % 8 == 0 and _CH % _L == 0
_UNROLL = 8

_mesh = plsc.VectorSubcoreMesh(
    core_axis_name="c", subcore_axis_name="s",
    num_cores=_NC, num_subcores=_NS)

_ABS_MASK = 0x7FFFFFFF
_INF_BITS = 0x7F800000


def _row_addr(k):
    q = k // 4
    m = k % 4
    b = m + 1 + jnp.where((m >= 2) & (q > 0), 1, 0)
    return q, b


@functools.partial(
    pl.kernel,
    out_type=jax.ShapeDtypeStruct((_NW, _L), jnp.int32),
    mesh=_mesh,
    scratch_types=[
        pltpu.VMEM((_CH,), jnp.float32),
        pltpu.VMEM((_CH,), jnp.float32),
        pltpu.VMEM((_CH,), jnp.float32),
        pltpu.VMEM((_CH,), jnp.float32),
        pltpu.VMEM((_L,), jnp.int32),
        pltpu.SemaphoreType.DMA,
        pltpu.SemaphoreType.DMA,
        pltpu.SemaphoreType.DMA,
        pltpu.SemaphoreType.DMA,
    ],
)
def _sc_masked_selfcmp(x_hbm, out_hbm, b0, b1, b2, b3, accv,
                       s0, s1, s2, s3):
    cid = lax.axis_index("c")
    sid = lax.axis_index("s")
    wid = sid * _NC + cid
    colbase = wid * _CW
    bufs = (b0, b1, b2, b3)
    sems = (s0, s1, s2, s3)

    absmask = jnp.full((_L,), _ABS_MASK, jnp.int32)

    def start(q, slot):
        a, b = _row_addr(q >> 1)
        cb = colbase + (q & 1) * _CH
        pltpu.async_copy(
            x_hbm.at[a, b, pl.ds(cb, _CH)], bufs[slot], sems[slot])

    def drain(slot):
        pltpu.make_async_copy(
            x_hbm.at[0, 1, pl.ds(colbase, _CH)],
            bufs[slot], sems[slot]).wait()

    def scan(slot, acc):
        buf = bufs[slot]

        def body(j, acc):
            base = j * (_L * _UNROLL)
            for u in range(_UNROLL):
                v = buf[pl.ds(base + u * _L, _L)]
                bits = lax.bitcast_convert_type(v, jnp.int32) & absmask
                acc = jnp.maximum(acc, bits)
            return acc
        return lax.fori_loop(0, _CH // (_L * _UNROLL), body, acc)

    for q in range(_NB - 1):
        start(jnp.int32(q), q)

    def step(g, acc):
        q0 = _NB * g
        for u in range(_NB):
            drain(u)

            @pl.when(q0 + u + (_NB - 1) < _NQ)
            def _(q0=q0, u=u):
                start(q0 + u + (_NB - 1), (u + _NB - 1) % _NB)

            acc = scan(u, acc)
        return acc

    acc = lax.fori_loop(0, _NQ // _NB, step, jnp.zeros((_L,), jnp.int32))

    accv[...] = acc
    pltpu.sync_copy(accv, out_hbm.at[wid])


def _tc_body(x_ref, out_ref):
    a = pl.program_id(0)
    j = pl.program_id(1)
    first = (a == 0) & (j == 0)
    ri = lax.broadcasted_iota(jnp.int32, (6, 1), 0)
    k0 = ((ri >= 1) & (ri <= 4)).astype(jnp.int32)
    k12 = ((ri != 0) & (ri != 3)).astype(jnp.int32)
    keep = jnp.where(a == 0, k0, k12)
    bits = lax.bitcast_convert_type(
        x_ref[0], jnp.int32) & jnp.int32(_ABS_MASK)
    masked = jnp.where(keep != 0, bits, 0)
    m6 = jnp.max(masked, axis=0)
    m = m6[0:_TCR]
    for u in range(1, _CT // _TCR):
        m = jnp.maximum(m, m6[u * _TCR:(u + 1) * _TCR])

    @pl.when(first)
    def _():
        out_ref[...] = m

    @pl.when(jnp.logical_not(first))
    def _():
        out_ref[...] = jnp.maximum(out_ref[...], m)


def _tc_scan(x):
    return pl.pallas_call(
        _tc_body,
        grid=(3, _NT),
        in_specs=[pl.BlockSpec(
            (1, 6, _CT), lambda a, j: (a, 0, (_CSC // _CT) + j))],
        out_specs=pl.BlockSpec((_TCR,), lambda a, j: (0,)),
        out_shape=jax.ShapeDtypeStruct((_TCR,), jnp.int32),
        compiler_params=pltpu.CompilerParams(
            dimension_semantics=("arbitrary", "arbitrary")),
    )(x)


def kernel(x):
    sc = _sc_masked_selfcmp(x)
    tc = _tc_scan(x)
    return jnp.maximum(jnp.max(sc), jnp.max(tc)) <= jnp.int32(_INF_BITS)


# speedup vs baseline: 1.4529x; 1.4529x over previous
import functools

import jax
import jax.numpy as jnp
from jax import lax
from jax.experimental import pallas as pl
from jax.experimental.pallas import tpu as pltpu
from jax.experimental.pallas import tpu_sc as plsc

_NC, _NS, _L = 2, 16, 16
_NW = _NC * _NS
_C = 1048576
_NROWS = 12

_CSC = 3 * _C // 4
_CTC = _C - _CSC
_CT = 32768
_NT = _CTC // _CT
_TCR = 2048

_CW = _CSC // _NW
_CH = _CW // 2
_NQ = 2 * _NROWS
_NB = 4
assert _CH % 8 == 0 and _CH % _L == 0
_UNROLL = 8

_mesh = plsc.VectorSubcoreMesh(
    core_axis_name="c", subcore_axis_name="s",
    num_cores=_NC, num_subcores=_NS)

_ABS_MASK = 0x7FFFFFFF
_INF_BITS = 0x7F800000


def _row_addr(k):
    q = k // 4
    m = k % 4
    b = m + 1 + jnp.where((m >= 2) & (q > 0), 1, 0)
    return q, b


@functools.partial(
    pl.kernel,
    out_type=jax.ShapeDtypeStruct((_NW, _L), jnp.int32),
    mesh=_mesh,
    scratch_types=[
        pltpu.VMEM((_CH,), jnp.float32),
        pltpu.VMEM((_CH,), jnp.float32),
        pltpu.VMEM((_CH,), jnp.float32),
        pltpu.VMEM((_CH,), jnp.float32),
        pltpu.VMEM((_L,), jnp.int32),
        pltpu.SemaphoreType.DMA,
        pltpu.SemaphoreType.DMA,
        pltpu.SemaphoreType.DMA,
        pltpu.SemaphoreType.DMA,
    ],
)
def _sc_masked_selfcmp(x_hbm, out_hbm, b0, b1, b2, b3, accv,
                       s0, s1, s2, s3):
    cid = lax.axis_index("c")
    sid = lax.axis_index("s")
    wid = sid * _NC + cid
    colbase = wid * _CW
    bufs = (b0, b1, b2, b3)
    sems = (s0, s1, s2, s3)

    absmask = jnp.full((_L,), _ABS_MASK, jnp.int32)

    def start(q, slot):
        a, b = _row_addr(q >> 1)
        cb = colbase + (q & 1) * _CH
        pltpu.async_copy(
            x_hbm.at[a, b, pl.ds(cb, _CH)], bufs[slot], sems[slot])

    def drain(slot):
        pltpu.make_async_copy(
            x_hbm.at[0, 1, pl.ds(colbase, _CH)],
            bufs[slot], sems[slot]).wait()

    def scan(slot, acc):
        buf = bufs[slot]

        def body(j, acc):
            base = j * (_L * _UNROLL)
            for u in range(_UNROLL):
                v = buf[pl.ds(base + u * _L, _L)]
                bits = lax.bitcast_convert_type(v, jnp.int32) & absmask
                acc = jnp.maximum(acc, bits)
            return acc
        return lax.fori_loop(0, _CH // (_L * _UNROLL), body, acc)

    for q in range(_NB - 1):
        start(jnp.int32(q), q)

    def step(g, acc):
        q0 = _NB * g
        for u in range(_NB):
            drain(u)

            @pl.when(q0 + u + (_NB - 1) < _NQ)
            def _(q0=q0, u=u):
                start(q0 + u + (_NB - 1), (u + _NB - 1) % _NB)

            acc = scan(u, acc)
        return acc

    acc = lax.fori_loop(0, _NQ // _NB, step, jnp.zeros((_L,), jnp.int32))

    accv[...] = acc
    pltpu.sync_copy(accv, out_hbm.at[wid])


def _tc_body(x_ref, out_ref):
    a = pl.program_id(1)
    first = a == 0
    ri = lax.broadcasted_iota(jnp.int32, (6, 1), 0)
    k0 = ((ri >= 1) & (ri <= 4)).astype(jnp.int32)
    k12 = ((ri != 0) & (ri != 3)).astype(jnp.int32)
    keep = jnp.where(a == 0, k0, k12) * jnp.int32(_ABS_MASK)
    masked = lax.bitcast_convert_type(x_ref[0], jnp.int32) & keep
    m6 = jnp.max(masked, axis=0)
    m = m6[0:_TCR]
    for u in range(1, _CT // _TCR):
        m = jnp.maximum(m, m6[u * _TCR:(u + 1) * _TCR])

    @pl.when(first)
    def _():
        out_ref[...] = m

    @pl.when(jnp.logical_not(first))
    def _():
        out_ref[...] = jnp.maximum(out_ref[...], m)


def _tc_scan(x):
    return pl.pallas_call(
        _tc_body,
        grid=(_NT, 3),
        in_specs=[pl.BlockSpec(
            (1, 6, _CT), lambda j, a: (a, 0, (_CSC // _CT) + j))],
        out_specs=pl.BlockSpec((_TCR,), lambda j, a: (j,)),
        out_shape=jax.ShapeDtypeStruct((_NT * _TCR,), jnp.int32),
        compiler_params=pltpu.CompilerParams(
            dimension_semantics=("parallel", "arbitrary")),
    )(x)


def kernel(x):
    sc = _sc_masked_selfcmp(x)
    tc = _tc_scan(x)
    return jnp.maximum(jnp.max(sc), jnp.max(tc)) <= jnp.int32(_INF_BITS)


# speedup vs baseline: 1.6881x; 1.1619x over previous
import functools

import jax
import jax.numpy as jnp
from jax import lax
from jax.experimental import pallas as pl
from jax.experimental.pallas import tpu as pltpu
from jax.experimental.pallas import tpu_sc as plsc

_NC, _NS, _L = 2, 16, 16
_NW = _NC * _NS
_C = 1048576
_NROWS = 12

_CW = _C // _NW
_CH = _CW // 2
_NQ = 2 * _NROWS
_NB = 4
assert _CH % 8 == 0 and _CH % _L == 0
_UNROLL = 8

_mesh = plsc.VectorSubcoreMesh(
    core_axis_name="c", subcore_axis_name="s",
    num_cores=_NC, num_subcores=_NS)

_ABS_MASK = 0x7FFFFFFF
_INF_BITS = 0x7F800000


def _row_addr(k):
    q = k // 4
    m = k % 4
    b = m + 1 + jnp.where((m >= 2) & (q > 0), 1, 0)
    return q, b


@functools.partial(
    pl.kernel,
    out_type=jax.ShapeDtypeStruct((_NW, _L), jnp.int32),
    mesh=_mesh,
    scratch_types=[
        pltpu.VMEM((_CH,), jnp.float32),
        pltpu.VMEM((_CH,), jnp.float32),
        pltpu.VMEM((_CH,), jnp.float32),
        pltpu.VMEM((_CH,), jnp.float32),
        pltpu.VMEM((_L,), jnp.int32),
        pltpu.SemaphoreType.DMA,
        pltpu.SemaphoreType.DMA,
        pltpu.SemaphoreType.DMA,
        pltpu.SemaphoreType.DMA,
    ],
)
def _sc_masked_selfcmp(x_hbm, out_hbm, b0, b1, b2, b3, accv,
                       s0, s1, s2, s3):
    cid = lax.axis_index("c")
    sid = lax.axis_index("s")
    wid = sid * _NC + cid
    colbase = wid * _CW
    bufs = (b0, b1, b2, b3)
    sems = (s0, s1, s2, s3)

    absmask = jnp.full((_L,), _ABS_MASK, jnp.int32)

    def start(q, slot):
        a, b = _row_addr(q >> 1)
        cb = colbase + (q & 1) * _CH
        pltpu.async_copy(
            x_hbm.at[a, b, pl.ds(cb, _CH)], bufs[slot], sems[slot])

    def drain(slot):
        pltpu.make_async_copy(
            x_hbm.at[0, 1, pl.ds(colbase, _CH)],
            bufs[slot], sems[slot]).wait()

    def scan(slot, acc):
        buf = bufs[slot]

        def body(j, acc):
            base = j * (_L * _UNROLL)
            for u in range(_UNROLL):
                v = buf[pl.ds(base + u * _L, _L)]
                bits = lax.bitcast_convert_type(v, jnp.int32) & absmask
                acc = jnp.maximum(acc, bits)
            return acc
        return lax.fori_loop(0, _CH // (_L * _UNROLL), body, acc)

    for q in range(_NB - 1):
        start(jnp.int32(q), q)

    def step(g, acc):
        q0 = _NB * g
        for u in range(_NB):
            drain(u)

            @pl.when(q0 + u + (_NB - 1) < _NQ)
            def _(q0=q0, u=u):
                start(q0 + u + (_NB - 1), (u + _NB - 1) % _NB)

            acc = scan(u, acc)
        return acc

    acc = lax.fori_loop(0, _NQ // _NB, step, jnp.zeros((_L,), jnp.int32))

    accv[...] = acc
    pltpu.sync_copy(accv, out_hbm.at[wid])


def kernel(x):
    partials = _sc_masked_selfcmp(x)
    return jnp.max(partials) <= jnp.int32(_INF_BITS)
